# baseline (speedup 1.0000x reference)
import jax
import jax.numpy as jnp
from jax import lax
from jax.experimental import pallas as pl
from jax.experimental.pallas import tpu as pltpu
from jax.experimental.pallas import tpu_sc as plsc

B, S, D = 4, 4096, 2048
NC, NS = 2, 16
NW = NC * NS
DW = 256
ND = D // DW
NV = DW // 16
S_CHUNK = 64
N_CHUNK = S // S_CHUNK
NB = 3


def _scan_chunk(in_ref, out_ref, carry):
    def rows(s2, carry):
        s = pl.multiple_of(s2 * 2, 2)
        for s_off in range(2):
            new = []
            for j in range(NV):
                x = in_ref[s + s_off, pl.ds(j * 16, 16)]
                acc = carry[j] + x
                out_ref[s + s_off, pl.ds(j * 16, 16)] = acc
                new.append(acc)
            carry = tuple(new)
        return carry

    return lax.fori_loop(0, S_CHUNK // 2, rows, carry)


def _cumsum_body(val_hbm, out_hbm, ins, outs, sls, sss):
    wid = lax.axis_index("s") * NC + lax.axis_index("c")
    b = wid // ND
    d0 = pl.multiple_of((wid % ND) * DW, DW)

    def hbm_in(c):
        return val_hbm.at[b, pl.ds(c * S_CHUNK, S_CHUNK), pl.ds(d0, DW)]

    def hbm_out(c):
        return out_hbm.at[b, pl.ds(c * S_CHUNK, S_CHUNK), pl.ds(d0, DW)]

    def start_load(c):
        pltpu.async_copy(hbm_in(c), ins[c % NB], sls[c % NB])

    def wait_load(c):
        pltpu.make_async_copy(hbm_in(c), ins[c % NB], sls[c % NB]).wait()

    def start_store(c):
        pltpu.async_copy(outs[c % NB], hbm_out(c), sss[c % NB])

    def wait_store(c):
        pltpu.make_async_copy(outs[c % NB], hbm_out(c), sss[c % NB]).wait()

    for c in range(NB - 1):
        start_load(c)
    carry = tuple(jnp.zeros((16,), jnp.float32) for _ in range(NV))
    for c in range(N_CHUNK):
        if c + NB - 1 < N_CHUNK:
            start_load(c + NB - 1)
        wait_load(c)
        if c >= NB:
            wait_store(c - NB)
        carry = _scan_chunk(ins[c % NB], outs[c % NB], carry)
        start_store(c)
    for c in range(N_CHUNK - NB, N_CHUNK):
        wait_store(c)


@jax.jit
def kernel(val):
    mesh = plsc.VectorSubcoreMesh(core_axis_name="c", subcore_axis_name="s")

    def body(val_hbm, out_hbm, i0, i1, i2, o0, o1, o2,
             sl0, sl1, sl2, ss0, ss1, ss2):
        _cumsum_body(val_hbm, out_hbm, (i0, i1, i2), (o0, o1, o2),
                     (sl0, sl1, sl2), (ss0, ss1, ss2))

    f = pl.kernel(
        body,
        out_type=jax.ShapeDtypeStruct((B, S, D), jnp.float32),
        mesh=mesh,
        scratch_types=(
            [pltpu.VMEM((S_CHUNK, DW), jnp.float32)] * (2 * NB)
            + [pltpu.SemaphoreType.DMA] * (2 * NB)
        ),
    )
    return f(val)


# speedup vs baseline: 1.0942x; 1.0942x over previous
import jax
import jax.numpy as jnp
from jax import lax
from jax.experimental import pallas as pl
from jax.experimental.pallas import tpu as pltpu
from jax.experimental.pallas import tpu_sc as plsc

B, S, D = 4, 4096, 2048
NC, NS = 2, 16
NW = NC * NS
DW = 256
ND = D // DW
NV = DW // 16
S_CHUNK = 32
N_CHUNK = S // S_CHUNK
NB = 4


def _scan_chunk(in_ref, out_ref, carry):
    def row(s, carry):
        new = []
        for j in range(NV):
            x = in_ref[s, pl.ds(j * 16, 16)]
            acc = carry[j] + x
            out_ref[s, pl.ds(j * 16, 16)] = acc
            new.append(acc)
        return tuple(new)

    return lax.fori_loop(0, S_CHUNK, row, carry)


def _cumsum_body(val_hbm, out_hbm, ins, outs, sls, sss):
    wid = lax.axis_index("s") * NC + lax.axis_index("c")
    b = wid // ND
    d0 = pl.multiple_of((wid % ND) * DW, DW)

    def hbm_in(c):
        s0 = pl.multiple_of(c * S_CHUNK, S_CHUNK)
        return val_hbm.at[b, pl.ds(s0, S_CHUNK), pl.ds(d0, DW)]

    def hbm_out(c):
        s0 = pl.multiple_of(c * S_CHUNK, S_CHUNK)
        return out_hbm.at[b, pl.ds(s0, S_CHUNK), pl.ds(d0, DW)]

    def start_load(c, k):
        pltpu.async_copy(hbm_in(c), ins[k], sls[k])

    def wait_load(c, k):
        pltpu.make_async_copy(hbm_in(c), ins[k], sls[k]).wait()

    def start_store(c, k):
        pltpu.async_copy(outs[k], hbm_out(c), sss[k])

    def wait_store(c, k):
        pltpu.make_async_copy(outs[k], hbm_out(c), sss[k]).wait()

    for c in range(NB - 1):
        start_load(c, c % NB)
    carry0 = tuple(jnp.zeros((16,), jnp.float32) for _ in range(NV))

    def group(i, carry):
        for k in range(NB):
            c = i * NB + k

            @pl.when(c + NB - 1 < N_CHUNK)
            def _():
                start_load(c + NB - 1, (k + NB - 1) % NB)

            wait_load(c, k)

            @pl.when(c >= NB)
            def _():
                wait_store(c - NB, k)

            carry = _scan_chunk(ins[k], outs[k], carry)
            start_store(c, k)
        return carry

    lax.fori_loop(0, N_CHUNK // NB, group, carry0)
    for c in range(N_CHUNK - NB, N_CHUNK):
        wait_store(c, c % NB)


@jax.jit
def kernel(val):
    mesh = plsc.VectorSubcoreMesh(core_axis_name="c", subcore_axis_name="s")

    def body(val_hbm, out_hbm, *scratch):
        _cumsum_body(val_hbm, out_hbm, scratch[:NB], scratch[NB:2 * NB],
                     scratch[2 * NB:3 * NB], scratch[3 * NB:])

    f = pl.kernel(
        body,
        out_type=jax.ShapeDtypeStruct((B, S, D), jnp.float32),
        mesh=mesh,
        scratch_types=(
            [pltpu.VMEM((S_CHUNK, DW), jnp.float32)] * (2 * NB)
            + [pltpu.SemaphoreType.DMA] * (2 * NB)
        ),
    )
    return f(val)


# speedup vs baseline: 1.1073x; 1.0120x over previous
import jax
import jax.numpy as jnp
from jax import lax
from jax.experimental import pallas as pl
from jax.experimental.pallas import tpu as pltpu
from jax.experimental.pallas import tpu_sc as plsc

B, S, D = 4, 4096, 2048
NC, NS = 2, 16
NW = NC * NS
DW = 256
ND = D // DW
NV = DW // 16
S_CHUNK = 64
N_CHUNK = S // S_CHUNK


def _scan_chunk(in_ref, out_ref, carry):
    def row(s, carry):
        new = []
        for j in range(NV):
            x = in_ref[s, pl.ds(j * 16, 16)]
            acc = carry[j] + x
            out_ref[s, pl.ds(j * 16, 16)] = acc
            new.append(acc)
        return tuple(new)

    return lax.fori_loop(0, S_CHUNK, row, carry)


def _cumsum_body(val_hbm, out_hbm, in0, in1, o0, o1, si0, si1, so0, so1):
    wid = lax.axis_index("s") * NC + lax.axis_index("c")
    b = wid // ND
    d0 = pl.multiple_of((wid % ND) * DW, DW)

    def src(c):
        s0 = pl.multiple_of(c * S_CHUNK, S_CHUNK)
        return val_hbm.at[b, pl.ds(s0, S_CHUNK), pl.ds(d0, DW)]

    def dst(c):
        s0 = pl.multiple_of(c * S_CHUNK, S_CHUNK)
        return out_hbm.at[b, pl.ds(s0, S_CHUNK), pl.ds(d0, DW)]

    pltpu.async_copy(src(0), in0, si0)
    carry0 = tuple(jnp.zeros((16,), jnp.float32) for _ in range(NV))

    def pair(i, carry):
        c0 = 2 * i
        pltpu.async_copy(src(c0 + 1), in1, si1)
        pltpu.make_async_copy(src(c0), in0, si0).wait()

        @pl.when(i > 0)
        def _():
            pltpu.make_async_copy(o0, dst(c0 - 2), so0).wait()

        carry = _scan_chunk(in0, o0, carry)
        pltpu.async_copy(o0, dst(c0), so0)

        @pl.when(c0 + 2 < N_CHUNK)
        def _():
            pltpu.async_copy(src(c0 + 2), in0, si0)

        pltpu.make_async_copy(src(c0 + 1), in1, si1).wait()

        @pl.when(i > 0)
        def _():
            pltpu.make_async_copy(o1, dst(c0 - 1), so1).wait()

        carry = _scan_chunk(in1, o1, carry)
        pltpu.async_copy(o1, dst(c0 + 1), so1)
        return carry

    lax.fori_loop(0, N_CHUNK // 2, pair, carry0)
    pltpu.make_async_copy(o0, dst(N_CHUNK - 2), so0).wait()
    pltpu.make_async_copy(o1, dst(N_CHUNK - 1), so1).wait()


@jax.jit
def kernel(val):
    mesh = plsc.VectorSubcoreMesh(core_axis_name="c", subcore_axis_name="s")
    f = pl.kernel(
        _cumsum_body,
        out_type=jax.ShapeDtypeStruct((B, S, D), jnp.float32),
        mesh=mesh,
        scratch_types=(
            [pltpu.VMEM((S_CHUNK, DW), jnp.float32)] * 4
            + [pltpu.SemaphoreType.DMA] * 4
        ),
    )
    return f(val)
